# Initial kernel scaffold; baseline (speedup 1.0000x reference)
import jax
import jax.numpy as jnp
from jax.experimental import pallas as pl


def kernel(b, conn, noise, bp_weights):
    raise NotImplementedError("write your pallas kernel here")


# speedup vs baseline: 2.1465x; 2.1465x over previous
import math

import numpy as np
import jax
import jax.numpy as jnp
from jax import lax
from jax.experimental import pallas as pl
from jax.experimental.pallas import tpu as pltpu

K = 500
NCHK = 500
N = 1000
M = 4
ITERS = 6
DC = 6
B = 1024
EBNO_DB = 5.0
NSYM = N // M
BB = 128

_SCALE = 1.0 / math.sqrt(10.0)
_NO = 1.0 / (10.0 ** (EBNO_DB / 10.0) * (K / N) * M)
_SIG = math.sqrt(_NO / 2.0)

_LAB = ((np.arange(16)[:, None] // np.array([8, 4, 2, 1])[None, :]) % 2)
_PR = (1.0 - 2.0 * _LAB[:, 0]) * (3.0 - 2.0 * _LAB[:, 1]) * _SCALE
_PI = (1.0 - 2.0 * _LAB[:, 2]) * (3.0 - 2.0 * _LAB[:, 3]) * _SCALE


def _dot(a, bm, dims):
    return lax.dot_general(a, bm, (dims, ((), ())),
                           precision=lax.Precision.HIGHEST,
                           preferred_element_type=jnp.float32)


def _mod2(x):
    return x - 2.0 * jnp.floor(x * 0.5)


def _body(bf_ref, nr_ref, ni_ref, at_ref, pa_ref, pb_ref, ptl_ref, ptr_ref,
          gl_ref, w_ref, out_ref):
    bfv = bf_ref[...]
    par = _mod2(_dot(bfv, at_ref[...], ((1,), (0,))))
    cbm = (_dot(bfv, pa_ref[...], ((1,), (0,)))
           + _dot(par, pb_ref[...], ((1,), (0,))))
    cb = [cbm[:, i * NSYM:(i + 1) * NSYM] for i in range(M)]
    xr = (1.0 - 2.0 * cb[0]) * (3.0 - 2.0 * cb[1]) * _SCALE
    xi = (1.0 - 2.0 * cb[2]) * (3.0 - 2.0 * cb[3]) * _SCALE
    yr = xr + nr_ref[...] * _SIG
    yi = xi + ni_ref[...] * _SIG
    d = [-((yr - float(_PR[p])) ** 2 + (yi - float(_PI[p])) ** 2) / _NO
         for p in range(16)]
    planes = []
    for i in range(M):
        lse = []
        for bit in (0, 1):
            s = [p for p in range(16) if _LAB[p, i] == bit]
            mx = d[s[0]]
            for p in s[1:]:
                mx = jnp.maximum(mx, d[p])
            acc = jnp.zeros_like(mx)
            for p in s:
                acc = acc + jnp.exp(d[p] - mx)
            lse.append(mx + jnp.log(acc))
        planes.append(lse[0] - lse[1])
    llr_l = jnp.zeros((BB, K), jnp.float32)
    llr_r = jnp.zeros((BB, NCHK), jnp.float32)
    for i in range(M):
        llr_l = llr_l + _dot(planes[i], ptl_ref[i * NSYM:(i + 1) * NSYM, :],
                             ((1,), (0,)))
        llr_r = llr_r + _dot(planes[i], ptr_ref[i * NSYM:(i + 1) * NSYM, :],
                             ((1,), (0,)))
    def check_update(gm, c2v, w):
        t, logt, negf = [], [], []
        for j in range(DC + 1):
            v2c = jnp.clip(gm[j] - c2v[j], -15.0, 15.0)
            tj = jnp.tanh(0.5 * v2c)
            t.append(tj)
            logt.append(jnp.log(jnp.abs(tj) + 1e-12))
            negf.append(jnp.where(tj < 0, 1.0, 0.0))
        sumlog = logt[0]
        nneg = negf[0]
        for j in range(1, DC + 1):
            sumlog = sumlog + logt[j]
            nneg = nneg + negf[j]
        sgn_tot = 1.0 - 2.0 * _mod2(nneg)
        new = []
        for j in range(DC + 1):
            excl_sgn = sgn_tot * (1.0 - 2.0 * negf[j])
            mag = jnp.clip(jnp.exp(sumlog - logt[j]), 0.0, 1.0 - 1e-7)
            new.append(w * excl_sgn * jnp.log((1.0 + mag) / (1.0 - mag)))
        return new

    glj = [gl_ref[:, j * NCHK:(j + 1) * NCHK] for j in range(DC)]
    gm = [_dot(llr_l, glj[j], ((1,), (0,))) for j in range(DC)] + [llr_r]
    c2v = [jnp.zeros((BB, NCHK), jnp.float32)] * (DC + 1)
    c2v = check_update(gm, c2v, w_ref[0, 0])
    for it in range(1, ITERS):
        marg_l = llr_l
        for j in range(DC):
            marg_l = marg_l + _dot(c2v[j], glj[j], ((1,), (1,)))
        marg_r = llr_r + c2v[DC]
        gm = [_dot(marg_l, glj[j], ((1,), (0,))) for j in range(DC)] + [marg_r]
        c2v = check_update(gm, c2v, w_ref[it, 0])
    outv = llr_l
    for j in range(DC):
        outv = outv + _dot(c2v[j], glj[j], ((1,), (1,)))
    out_ref[...] = outv


def kernel(b, conn, noise, bp_weights):
    bf = b.astype(jnp.float32)
    nr = noise[..., 0]
    ni = noise[..., 1]
    vk = jnp.arange(K, dtype=jnp.int32)
    at = (conn[None, :, :] == vk[:, None, None]).sum(-1).astype(jnp.float32)
    gl = (vk[:, None, None] == conn.T[None, :, :]).astype(jnp.float32)
    gl = gl.reshape(K, DC * NCHK)
    cols = np.arange(N)
    n_of_col = M * (cols % NSYM) + cols // NSYM
    pmat = (np.arange(N)[:, None] == n_of_col[None, :]).astype(np.float32)
    pa = jnp.asarray(pmat[:K, :])
    pb = jnp.asarray(pmat[K:, :])
    ptl = jnp.asarray(pmat.T[:, :K])
    ptr = jnp.asarray(pmat.T[:, K:])
    w2 = bp_weights.reshape(ITERS, 1)

    grid = (B // BB,)
    full = lambda shape: pl.BlockSpec(shape, lambda i: (0, 0))
    blk = lambda shape: pl.BlockSpec(shape, lambda i: (i, 0))
    out = pl.pallas_call(
        _body,
        grid=grid,
        in_specs=[
            blk((BB, K)),
            blk((BB, NSYM)),
            blk((BB, NSYM)),
            full((K, NCHK)),
            full((K, N)),
            full((NCHK, N)),
            full((N, K)),
            full((N, NCHK)),
            full((K, DC * NCHK)),
            pl.BlockSpec(memory_space=pltpu.SMEM),
        ],
        out_specs=blk((BB, K)),
        out_shape=jax.ShapeDtypeStruct((B, K), jnp.float32),
    )(bf, nr, ni, at, pa, pb, ptl, ptr, gl, w2)
    return out


# speedup vs baseline: 2.4817x; 1.1562x over previous
import math

import numpy as np
import jax
import jax.numpy as jnp
from jax import lax
from jax.experimental import pallas as pl
from jax.experimental.pallas import tpu as pltpu

K = 500
NCHK = 500
N = 1000
M = 4
ITERS = 6
DC = 6
B = 1024
EBNO_DB = 5.0
NSYM = N // M
BB = 128

_SCALE = 1.0 / math.sqrt(10.0)
_NO = 1.0 / (10.0 ** (EBNO_DB / 10.0) * (K / N) * M)
_SIG = math.sqrt(_NO / 2.0)

_LAB = ((np.arange(16)[:, None] // np.array([8, 4, 2, 1])[None, :]) % 2)
_PR = (1.0 - 2.0 * _LAB[:, 0]) * (3.0 - 2.0 * _LAB[:, 1]) * _SCALE
_PI = (1.0 - 2.0 * _LAB[:, 2]) * (3.0 - 2.0 * _LAB[:, 3]) * _SCALE


def _dot(a, bm, dims, precision=lax.Precision.HIGHEST):
    return lax.dot_general(a, bm, (dims, ((), ())),
                           precision=precision,
                           preferred_element_type=jnp.float32)


def _dot_exact01(a, bm, dims):
    return _dot(a, bm, dims, precision=lax.Precision.DEFAULT)


def _gather3(x, gb, dims):
    f32, bf16 = jnp.float32, jnp.bfloat16
    hi_b = x.astype(bf16)
    r = x - hi_b.astype(f32)
    mid_b = r.astype(bf16)
    lo_b = (r - mid_b.astype(f32)).astype(bf16)
    out = _dot(hi_b, gb, dims, precision=lax.Precision.DEFAULT)
    out = out + _dot(mid_b, gb, dims, precision=lax.Precision.DEFAULT)
    out = out + _dot(lo_b, gb, dims, precision=lax.Precision.DEFAULT)
    return out


def _mod2(x):
    return x - 2.0 * jnp.floor(x * 0.5)


def _body(bf_ref, nr_ref, ni_ref, at_ref, pa_ref, pb_ref, ptl_ref, ptr_ref,
          gl_ref, glb_ref, w_ref, out_ref):
    bfv = bf_ref[...]
    par = _mod2(_dot_exact01(bfv, at_ref[...], ((1,), (0,))))
    cbm = (_dot_exact01(bfv, pa_ref[...], ((1,), (0,)))
           + _dot_exact01(par, pb_ref[...], ((1,), (0,))))
    cb = [cbm[:, i * NSYM:(i + 1) * NSYM] for i in range(M)]
    xr = (1.0 - 2.0 * cb[0]) * (3.0 - 2.0 * cb[1]) * _SCALE
    xi = (1.0 - 2.0 * cb[2]) * (3.0 - 2.0 * cb[3]) * _SCALE
    yr = xr + nr_ref[...] * _SIG
    yi = xi + ni_ref[...] * _SIG
    d = [-((yr - float(_PR[p])) ** 2 + (yi - float(_PI[p])) ** 2) / _NO
         for p in range(16)]
    planes = []
    for i in range(M):
        lse = []
        for bit in (0, 1):
            s = [p for p in range(16) if _LAB[p, i] == bit]
            mx = d[s[0]]
            for p in s[1:]:
                mx = jnp.maximum(mx, d[p])
            acc = jnp.zeros_like(mx)
            for p in s:
                acc = acc + jnp.exp(d[p] - mx)
            lse.append(mx + jnp.log(acc))
        planes.append(lse[0] - lse[1])
    pcat = jnp.concatenate(planes, axis=1)
    llr_l = _gather3(pcat, ptl_ref[...], ((1,), (0,)))
    llr_r = _gather3(pcat, ptr_ref[...], ((1,), (0,)))
    def check_update(gm, c2v, w):
        t, logt, negf = [], [], []
        for j in range(DC + 1):
            v2c = jnp.clip(gm[j] - c2v[j], -15.0, 15.0)
            tj = jnp.tanh(0.5 * v2c)
            t.append(tj)
            logt.append(jnp.log(jnp.abs(tj) + 1e-12))
            negf.append(jnp.where(tj < 0, 1.0, 0.0))
        sumlog = logt[0]
        nneg = negf[0]
        for j in range(1, DC + 1):
            sumlog = sumlog + logt[j]
            nneg = nneg + negf[j]
        sgn_tot = 1.0 - 2.0 * _mod2(nneg)
        new = []
        for j in range(DC + 1):
            excl_sgn = sgn_tot * (1.0 - 2.0 * negf[j])
            mag = jnp.clip(jnp.exp(sumlog - logt[j]), 0.0, 1.0 - 1e-7)
            new.append(w * excl_sgn * jnp.log((1.0 + mag) / (1.0 - mag)))
        return new

    glj = [gl_ref[:, j * NCHK:(j + 1) * NCHK] for j in range(DC)]

    def gather_all(m_l, m_r):
        gcat = _gather3(m_l, glb_ref[...], ((1,), (0,)))
        return [gcat[:, j * NCHK:(j + 1) * NCHK] for j in range(DC)] + [m_r]

    gm = gather_all(llr_l, llr_r)
    c2v = [jnp.zeros((BB, NCHK), jnp.float32)] * (DC + 1)
    c2v = check_update(gm, c2v, w_ref[0, 0])
    for it in range(1, ITERS):
        marg_l = llr_l
        for j in range(DC):
            marg_l = marg_l + _dot(c2v[j], glj[j], ((1,), (1,)))
        marg_r = llr_r + c2v[DC]
        gm = gather_all(marg_l, marg_r)
        c2v = check_update(gm, c2v, w_ref[it, 0])
    outv = llr_l
    for j in range(DC):
        outv = outv + _dot(c2v[j], glj[j], ((1,), (1,)))
    out_ref[...] = outv


def kernel(b, conn, noise, bp_weights):
    bf = b.astype(jnp.float32)
    nr = noise[..., 0]
    ni = noise[..., 1]
    vk = jnp.arange(K, dtype=jnp.int32)
    at = (conn[None, :, :] == vk[:, None, None]).sum(-1).astype(jnp.float32)
    gl = (vk[:, None, None] == conn.T[None, :, :]).astype(jnp.float32)
    gl = gl.reshape(K, DC * NCHK)
    cols = np.arange(N)
    n_of_col = M * (cols % NSYM) + cols // NSYM
    pmat = (np.arange(N)[:, None] == n_of_col[None, :]).astype(np.float32)
    pa = jnp.asarray(pmat[:K, :])
    pb = jnp.asarray(pmat[K:, :])
    ptl = jnp.asarray(pmat.T[:, :K]).astype(jnp.bfloat16)
    ptr = jnp.asarray(pmat.T[:, K:]).astype(jnp.bfloat16)
    glb = gl.astype(jnp.bfloat16)
    w2 = bp_weights.reshape(ITERS, 1)

    grid = (B // BB,)
    full = lambda shape: pl.BlockSpec(shape, lambda i: (0, 0))
    blk = lambda shape: pl.BlockSpec(shape, lambda i: (i, 0))
    out = pl.pallas_call(
        _body,
        grid=grid,
        in_specs=[
            blk((BB, K)),
            blk((BB, NSYM)),
            blk((BB, NSYM)),
            full((K, NCHK)),
            full((K, N)),
            full((NCHK, N)),
            full((N, K)),
            full((N, NCHK)),
            full((K, DC * NCHK)),
            full((K, DC * NCHK)),
            pl.BlockSpec(memory_space=pltpu.SMEM),
        ],
        out_specs=blk((BB, K)),
        out_shape=jax.ShapeDtypeStruct((B, K), jnp.float32),
    )(bf, nr, ni, at, pa, pb, ptl, ptr, gl, glb, w2)
    return out
